# baseline (speedup 1.0000x reference)
import functools

import jax
import jax.numpy as jnp
from jax import lax
from jax.experimental import pallas as pl
from jax.experimental.pallas import tpu as pltpu
from jax.experimental.pallas import tpu_sc as plsc

NC = 2
NS = 16
NW = NC * NS
LANES = 16


@functools.lru_cache(maxsize=None)
def _make_kernel(batch, maxlen, embed, vocab):
    assert embed == 2 * LANES and maxlen % 4 == 0 and vocab % 8 == 0
    spw = batch // NW
    half = embed // 2

    ngrp = vocab // 8
    g1 = 16
    nch = -(-ngrp // g1)
    per_w = -(-nch // NS)

    mesh = plsc.VectorSubcoreMesh(core_axis_name="c", subcore_axis_name="s")

    @functools.partial(
        pl.kernel,
        out_type=jax.ShapeDtypeStruct((batch, maxlen, embed), jnp.float32),
        mesh=mesh,
        scratch_types=[
            pltpu.HBM((vocab, embed), jnp.float32),
            pltpu.VMEM((g1, 8, embed), jnp.float32),
            pltpu.VMEM((g1, 8, embed), jnp.float32),
            pltpu.VMEM((maxlen,), jnp.int32),
            pltpu.VMEM((maxlen,), jnp.int32),
            pltpu.VMEM((maxlen, embed), jnp.float32),
            pltpu.VMEM((maxlen, embed), jnp.float32),
            pltpu.VMEM((maxlen // 4, 4 * embed), jnp.float32),
            pltpu.SemaphoreType.DMA,
            pltpu.SemaphoreType.DMA,
            pltpu.SemaphoreType.DMA,
            pltpu.SemaphoreType.DMA,
            pltpu.SemaphoreType.DMA,
            pltpu.SemaphoreType.DMA,
        ],
    )
    def k(x_hbm, tab_hbm, pos_hbm, out_hbm,
          ctab, tb0, tb1, ib0, ib1, rows0, rows1, pos_v,
          s0, s1, s2, s3, s4, s5):
        sid = lax.axis_index("s")
        wid = sid * NC + lax.axis_index("c")

        tbufs = (tb0, tb1)
        si = (s0, s1)
        so = (s2, s3)

        def goff(i):
            return jnp.minimum((sid + i * NS) * g1, ngrp - g1)

        def active(i):
            return sid + i * NS < nch

        def detile_body(c, carry):
            @pl.when(active(c))
            def _emit():
                pltpu.sync_copy(tab_hbm.at[pl.ds(goff(c), g1)], tb0)
                for j in range(g1):
                    pltpu.async_copy(
                        tb0.at[j],
                        ctab.at[pl.ds((goff(c) + j) * 8, 8)], so[0])
                for j in range(g1):
                    pltpu.make_async_copy(
                        tb0.at[j],
                        ctab.at[pl.ds((goff(c) + j) * 8, 8)], so[0]).wait()
            return carry

        lax.fori_loop(0, per_w, detile_body, 0)

        plsc.subcore_barrier()

        sbase = wid * spw
        ib = (ib0, ib1)
        bufs = (rows0, rows1)
        gi_ = (s0, s1)
        gg = (s2, s3)
        go = (s4, s5)

        pltpu.sync_copy(pos_hbm, pos_v)
        pltpu.sync_copy(x_hbm.at[sbase], ib0)
        pltpu.async_copy(ctab.at[ib0], rows0, s2)
        pltpu.async_copy(x_hbm.at[sbase + 1], ib1, s1)

        def pair_body(gi, carry):
            for b in range(2):
                i = 2 * gi + b
                rb, rnb = bufs[b], bufs[1 - b]

                @pl.when(i >= 1)
                def _wait_prev_out():
                    pltpu.make_async_copy(
                        rnb.reshape(1, maxlen, embed),
                        out_hbm.at[pl.ds(sbase + i - 1, 1)],
                        go[1 - b]).wait()

                @pl.when(i + 1 < spw)
                def _fire_next_gather():
                    pltpu.make_async_copy(
                        x_hbm.at[sbase + i + 1], ib[1 - b], gi_[1 - b]).wait()
                    pltpu.async_copy(ctab.at[ib[1 - b]], rnb, gg[1 - b])

                pltpu.make_async_copy(ctab.at[ib[b]], rb, gg[b]).wait()

                @pl.when(i + 2 < spw)
                def _fetch_next_idx():
                    pltpu.async_copy(x_hbm.at[sbase + i + 2], ib[b], gi_[b])

                @plsc.parallel_loop(0, maxlen, unroll=2)
                def _add_t(t):
                    prow = t >> 2
                    pcol = (t & 3) * embed
                    p0 = pos_v[prow, pl.ds(pcol, LANES)]
                    p1 = pos_v[prow, pl.ds(pcol + half, LANES)]
                    rb[t, pl.ds(0, LANES)] = rb[t, pl.ds(0, LANES)] + p0
                    rb[t, pl.ds(half, LANES)] = rb[t, pl.ds(half, LANES)] + p1

                pltpu.async_copy(
                    rb.reshape(1, maxlen, embed),
                    out_hbm.at[pl.ds(sbase + i, 1)], go[b])
            return carry

        lax.fori_loop(0, spw // 2, pair_body, 0)
        pltpu.make_async_copy(
            bufs[1].reshape(1, maxlen, embed),
            out_hbm.at[pl.ds(sbase + spw - 1, 1)], go[1]).wait()

    return k


def kernel(x, token_table, pos_table):
    batch, maxlen = x.shape
    vocab, embed = token_table.shape
    tab3 = token_table.reshape(vocab // 8, 8, embed)
    pos128 = pos_table.reshape(maxlen // 4, 4 * embed)
    xi = x.astype(jnp.int32)
    return _make_kernel(batch, maxlen, embed, vocab)(xi, tab3, pos128)


# speedup vs baseline: 1.2579x; 1.2579x over previous
import functools

import jax
import jax.numpy as jnp
from jax import lax
from jax.experimental import pallas as pl
from jax.experimental.pallas import tpu as pltpu
from jax.experimental.pallas import tpu_sc as plsc

NC = 2
NS = 16
NW = NC * NS
LANES = 16


@functools.lru_cache(maxsize=None)
def _make_kernel(batch, maxlen, embed, vocab):
    assert embed == 2 * LANES and maxlen % 4 == 0 and vocab % 8 == 0
    spw = batch // NW
    half = embed // 2

    ngrp = vocab // 8
    g1 = 16
    nch = -(-ngrp // g1)
    per_w = -(-nch // NS)

    mesh = plsc.VectorSubcoreMesh(core_axis_name="c", subcore_axis_name="s")

    @functools.partial(
        pl.kernel,
        out_type=jax.ShapeDtypeStruct((batch, maxlen, embed), jnp.float32),
        mesh=mesh,
        scratch_types=[
            pltpu.HBM((vocab, embed), jnp.float32),
            pltpu.VMEM((g1, 8, embed), jnp.float32),
            pltpu.VMEM((g1, 8, embed), jnp.float32),
            pltpu.VMEM((g1 * 8, embed), jnp.float32),
            pltpu.VMEM((g1 * 8, embed), jnp.float32),
            pltpu.VMEM((maxlen,), jnp.int32),
            pltpu.VMEM((maxlen,), jnp.int32),
            pltpu.VMEM((maxlen, embed), jnp.float32),
            pltpu.VMEM((maxlen, embed), jnp.float32),
            pltpu.VMEM((maxlen // 4, 4 * embed), jnp.float32),
            pltpu.SemaphoreType.DMA,
            pltpu.SemaphoreType.DMA,
            pltpu.SemaphoreType.DMA,
            pltpu.SemaphoreType.DMA,
            pltpu.SemaphoreType.DMA,
            pltpu.SemaphoreType.DMA,
        ],
    )
    def k(x_hbm, tab_hbm, pos_hbm, out_hbm,
          ctab, tb0, tb1, wb0, wb1, ib0, ib1, rows0, rows1, pos_v,
          s0, s1, s2, s3, s4, s5):
        sid = lax.axis_index("s")
        wid = sid * NC + lax.axis_index("c")

        tbufs = (tb0, tb1)
        wbufs = (wb0, wb1)
        si = (s0, s1)
        so = (s2, s3)

        def goff(i):
            return jnp.minimum((sid + i * NS) * g1, ngrp - g1)

        def active(i):
            return sid + i * NS < nch

        pltpu.async_copy(tab_hbm.at[pl.ds(goff(0), g1)], tb0, s0)

        def detile_body(c, carry):
            for b in range(2):
                i = 2 * c + b
                ab, wb = tbufs[b], wbufs[b]

                @pl.when((i >= 1) & active(i - 1) & active(i))
                def _drain_prev():
                    pltpu.make_async_copy(
                        wbufs[1 - b],
                        ctab.at[pl.ds(goff(i - 1) * 8, g1 * 8)],
                        so[1 - b]).wait()

                @pl.when(active(i + 1))
                def _fire_next():
                    pltpu.async_copy(
                        tab_hbm.at[pl.ds(goff(i + 1), g1)],
                        tbufs[1 - b], si[1 - b])

                @pl.when(active(i))
                def _emit():
                    pltpu.make_async_copy(
                        tab_hbm.at[pl.ds(goff(i), g1)], ab, si[b]).wait()

                    @plsc.parallel_loop(0, g1, unroll=2)
                    def _regcopy(g):
                        for j in range(8):
                            wb[g * 8 + j, pl.ds(0, LANES)] = (
                                ab[g, j, pl.ds(0, LANES)])
                            wb[g * 8 + j, pl.ds(LANES, LANES)] = (
                                ab[g, j, pl.ds(LANES, LANES)])

                    pltpu.async_copy(
                        wb, ctab.at[pl.ds(goff(i) * 8, g1 * 8)], so[b])
            return carry

        lax.fori_loop(0, -(-per_w // 2), detile_body, 0)
        lw = (nch - 1 - sid) // NS

        for b in range(2):
            @pl.when((lw % 2) == b)
            def _drain_last():
                pltpu.make_async_copy(
                    wbufs[b],
                    ctab.at[pl.ds(goff(lw) * 8, g1 * 8)], so[b]).wait()

        plsc.subcore_barrier()

        sbase = wid * spw
        ib = (ib0, ib1)
        bufs = (rows0, rows1)
        gi_ = (s0, s1)
        gg = (s2, s3)
        go = (s4, s5)

        pltpu.sync_copy(pos_hbm, pos_v)
        pltpu.sync_copy(x_hbm.at[sbase], ib0)
        pltpu.async_copy(ctab.at[ib0], rows0, s2)
        pltpu.async_copy(x_hbm.at[sbase + 1], ib1, s1)

        def pair_body(gi, carry):
            for b in range(2):
                i = 2 * gi + b
                rb, rnb = bufs[b], bufs[1 - b]

                @pl.when(i >= 1)
                def _wait_prev_out():
                    pltpu.make_async_copy(
                        rnb.reshape(1, maxlen, embed),
                        out_hbm.at[pl.ds(sbase + i - 1, 1)],
                        go[1 - b]).wait()

                @pl.when(i + 1 < spw)
                def _fire_next_gather():
                    pltpu.make_async_copy(
                        x_hbm.at[sbase + i + 1], ib[1 - b], gi_[1 - b]).wait()
                    pltpu.async_copy(ctab.at[ib[1 - b]], rnb, gg[1 - b])

                pltpu.make_async_copy(ctab.at[ib[b]], rb, gg[b]).wait()

                @pl.when(i + 2 < spw)
                def _fetch_next_idx():
                    pltpu.async_copy(x_hbm.at[sbase + i + 2], ib[b], gi_[b])

                @plsc.parallel_loop(0, maxlen, unroll=2)
                def _add_t(t):
                    prow = t >> 2
                    pcol = (t & 3) * embed
                    p0 = pos_v[prow, pl.ds(pcol, LANES)]
                    p1 = pos_v[prow, pl.ds(pcol + half, LANES)]
                    rb[t, pl.ds(0, LANES)] = rb[t, pl.ds(0, LANES)] + p0
                    rb[t, pl.ds(half, LANES)] = rb[t, pl.ds(half, LANES)] + p1

                pltpu.async_copy(
                    rb.reshape(1, maxlen, embed),
                    out_hbm.at[pl.ds(sbase + i, 1)], go[b])
            return carry

        lax.fori_loop(0, spw // 2, pair_body, 0)
        pltpu.make_async_copy(
            bufs[1].reshape(1, maxlen, embed),
            out_hbm.at[pl.ds(sbase + spw - 1, 1)], go[1]).wait()

    return k


def kernel(x, token_table, pos_table):
    batch, maxlen = x.shape
    vocab, embed = token_table.shape
    tab3 = token_table.reshape(vocab // 8, 8, embed)
    pos128 = pos_table.reshape(maxlen // 4, 4 * embed)
    xi = x.astype(jnp.int32)
    return _make_kernel(batch, maxlen, embed, vocab)(xi, tab3, pos128)


# speedup vs baseline: 1.4777x; 1.1748x over previous
import functools

import jax
import jax.numpy as jnp
from jax import lax
from jax.experimental import pallas as pl
from jax.experimental.pallas import tpu as pltpu
from jax.experimental.pallas import tpu_sc as plsc

NC = 2
NS = 16
NW = NC * NS
LANES = 16


@functools.lru_cache(maxsize=None)
def _make_kernel(batch, maxlen, embed, vocab):
    assert embed == 2 * LANES and maxlen % 4 == 0 and vocab % 8 == 0
    spw = batch // NW
    half = embed // 2

    ngrp = vocab // 8
    g1 = 16
    nch = -(-ngrp // g1)
    per_w = -(-nch // NS)

    mesh = plsc.VectorSubcoreMesh(core_axis_name="c", subcore_axis_name="s")

    @functools.partial(
        pl.kernel,
        out_type=jax.ShapeDtypeStruct((batch, maxlen, embed), jnp.float32),
        mesh=mesh,
        scratch_types=[
            pltpu.HBM((vocab, embed), jnp.float32),
            pltpu.VMEM((g1, 8, embed), jnp.float32),
            pltpu.VMEM((g1, 8, embed), jnp.float32),
            pltpu.VMEM((g1 * 8, embed), jnp.float32),
            pltpu.VMEM((g1 * 8, embed), jnp.float32),
            pltpu.VMEM((maxlen,), jnp.int32),
            pltpu.VMEM((maxlen,), jnp.int32),
            pltpu.VMEM((maxlen, embed), jnp.float32),
            pltpu.VMEM((maxlen, embed), jnp.float32),
            pltpu.VMEM((maxlen // 4, 4 * embed), jnp.float32),
            pltpu.SemaphoreType.DMA,
            pltpu.SemaphoreType.DMA,
            pltpu.SemaphoreType.DMA,
            pltpu.SemaphoreType.DMA,
            pltpu.SemaphoreType.DMA,
            pltpu.SemaphoreType.DMA,
        ],
    )
    def k(x_hbm, tab_hbm, pos_hbm, out_hbm,
          ctab, tb0, tb1, wb0, wb1, ib0, ib1, rows0, rows1, pos_v,
          s0, s1, s2, s3, s4, s5):
        sid = lax.axis_index("s")
        wid = sid * NC + lax.axis_index("c")

        tbufs = (tb0, tb1)
        wbufs = (wb0, wb1)
        si = (s0, s1)
        so = (s2, s3)

        def goff(i):
            return jnp.minimum((sid + i * NS) * g1, ngrp - g1)

        def active(i):
            return sid + i * NS < nch

        pltpu.async_copy(tab_hbm.at[pl.ds(goff(0), g1)], tb0, s0)

        def detile_body(c, carry):
            for b in range(2):
                i = 2 * c + b
                ab, wb = tbufs[b], wbufs[b]

                @pl.when((i >= 1) & active(i - 1) & active(i))
                def _drain_prev():
                    pltpu.make_async_copy(
                        wbufs[1 - b],
                        ctab.at[pl.ds(goff(i - 1) * 8, g1 * 8)],
                        so[1 - b]).wait()

                @pl.when(active(i + 1))
                def _fire_next():
                    pltpu.async_copy(
                        tab_hbm.at[pl.ds(goff(i + 1), g1)],
                        tbufs[1 - b], si[1 - b])

                @pl.when(active(i))
                def _emit():
                    pltpu.make_async_copy(
                        tab_hbm.at[pl.ds(goff(i), g1)], ab, si[b]).wait()

                    @plsc.parallel_loop(0, g1, unroll=2)
                    def _regcopy(g):
                        for j in range(8):
                            wb[g * 8 + j, pl.ds(0, LANES)] = (
                                ab[g, j, pl.ds(0, LANES)])
                            wb[g * 8 + j, pl.ds(LANES, LANES)] = (
                                ab[g, j, pl.ds(LANES, LANES)])

                    pltpu.async_copy(
                        wb, ctab.at[pl.ds(goff(i) * 8, g1 * 8)], so[b])
            return carry

        lax.fori_loop(0, -(-per_w // 2), detile_body, 0)
        lw = (nch - 1 - sid) // NS

        for b in range(2):
            @pl.when((lw % 2) == b)
            def _drain_last():
                pltpu.make_async_copy(
                    wbufs[b],
                    ctab.at[pl.ds(goff(lw) * 8, g1 * 8)], so[b]).wait()

        plsc.subcore_barrier()
        return

        sbase = wid * spw
        ib = (ib0, ib1)
        bufs = (rows0, rows1)
        gi_ = (s0, s1)
        gg = (s2, s3)
        go = (s4, s5)

        pltpu.sync_copy(pos_hbm, pos_v)
        pltpu.sync_copy(x_hbm.at[sbase], ib0)
        pltpu.async_copy(ctab.at[ib0], rows0, s2)
        pltpu.async_copy(x_hbm.at[sbase + 1], ib1, s1)

        def pair_body(gi, carry):
            for b in range(2):
                i = 2 * gi + b
                rb, rnb = bufs[b], bufs[1 - b]

                @pl.when(i >= 1)
                def _wait_prev_out():
                    pltpu.make_async_copy(
                        rnb.reshape(1, maxlen, embed),
                        out_hbm.at[pl.ds(sbase + i - 1, 1)],
                        go[1 - b]).wait()

                @pl.when(i + 1 < spw)
                def _fire_next_gather():
                    pltpu.make_async_copy(
                        x_hbm.at[sbase + i + 1], ib[1 - b], gi_[1 - b]).wait()
                    pltpu.async_copy(ctab.at[ib[1 - b]], rnb, gg[1 - b])

                pltpu.make_async_copy(ctab.at[ib[b]], rb, gg[b]).wait()

                @pl.when(i + 2 < spw)
                def _fetch_next_idx():
                    pltpu.async_copy(x_hbm.at[sbase + i + 2], ib[b], gi_[b])

                @plsc.parallel_loop(0, maxlen, unroll=2)
                def _add_t(t):
                    prow = t >> 2
                    pcol = (t & 3) * embed
                    p0 = pos_v[prow, pl.ds(pcol, LANES)]
                    p1 = pos_v[prow, pl.ds(pcol + half, LANES)]
                    rb[t, pl.ds(0, LANES)] = rb[t, pl.ds(0, LANES)] + p0
                    rb[t, pl.ds(half, LANES)] = rb[t, pl.ds(half, LANES)] + p1

                pltpu.async_copy(
                    rb.reshape(1, maxlen, embed),
                    out_hbm.at[pl.ds(sbase + i, 1)], go[b])
            return carry

        lax.fori_loop(0, spw // 2, pair_body, 0)
        pltpu.make_async_copy(
            bufs[1].reshape(1, maxlen, embed),
            out_hbm.at[pl.ds(sbase + spw - 1, 1)], go[1]).wait()

    return k


def kernel(x, token_table, pos_table):
    batch, maxlen = x.shape
    vocab, embed = token_table.shape
    tab3 = token_table.reshape(vocab // 8, 8, embed)
    pos128 = pos_table.reshape(maxlen // 4, 4 * embed)
    xi = x.astype(jnp.int32)
    return _make_kernel(batch, maxlen, embed, vocab)(xi, tab3, pos128)


# speedup vs baseline: 1.6864x; 1.1412x over previous
import functools

import jax
import jax.numpy as jnp
from jax import lax
from jax.experimental import pallas as pl
from jax.experimental.pallas import tpu as pltpu
from jax.experimental.pallas import tpu_sc as plsc

NC = 2
NS = 16
NW = NC * NS
LANES = 16
SEQS_PER_CHUNK = 4


@functools.lru_cache(maxsize=None)
def _make_kernel(batch, maxlen, embed, vocab):
    rows = batch * maxlen
    assert rows % NW == 0
    rpw = rows // NW
    ch = SEQS_PER_CHUNK * maxlen
    assert rpw % (2 * ch) == 0
    nchunk = rpw // ch
    half = embed // 2
    assert embed == 2 * LANES

    mesh = plsc.VectorSubcoreMesh(core_axis_name="c", subcore_axis_name="s")

    @functools.partial(
        pl.kernel,
        out_type=jax.ShapeDtypeStruct((batch, maxlen, embed), jnp.float32),
        mesh=mesh,
        scratch_types=[
            pltpu.VMEM((ch,), jnp.int32),
            pltpu.VMEM((ch,), jnp.int32),
            pltpu.VMEM((ch, embed), jnp.float32),
            pltpu.VMEM((ch, embed), jnp.float32),
            pltpu.VMEM((maxlen, embed), jnp.float32),
            pltpu.SemaphoreType.DMA,
            pltpu.SemaphoreType.DMA,
            pltpu.SemaphoreType.DMA,
            pltpu.SemaphoreType.DMA,
            pltpu.SemaphoreType.DMA,
            pltpu.SemaphoreType.DMA,
        ],
        compiler_params=pltpu.CompilerParams(use_tc_tiling_on_sc=False),
    )
    def k(x_hbm, tok_hbm, pos_hbm, out_hbm,
          ib0, ib1, rows0, rows1, pos_v,
          si0, si1, sg0, sg1, so0, so1):
        wid = lax.axis_index("s") * NC + lax.axis_index("c")
        base = wid * rpw
        sq0 = wid * (rpw // maxlen)
        ib = (ib0, ib1)
        bufs = (rows0, rows1)
        si = (si0, si1)
        sg = (sg0, sg1)
        so = (so0, so1)

        pltpu.sync_copy(pos_hbm, pos_v)
        pltpu.sync_copy(x_hbm.at[wid, 0], ib0)
        pltpu.async_copy(tok_hbm.at[ib0], rows0, sg0)
        pltpu.async_copy(x_hbm.at[wid, 1], ib1, si1)

        def pair_body(gi, carry):
            for b in range(2):
                i = 2 * gi + b
                rb, rnb = bufs[b], bufs[1 - b]

                @pl.when(i >= 1)
                def _wait_prev_out():
                    for kq in range(SEQS_PER_CHUNK):
                        pltpu.make_async_copy(
                            rnb.at[pl.ds(kq * maxlen, maxlen)],
                            out_hbm.at[sq0 + (i - 1) * SEQS_PER_CHUNK + kq],
                            so[1 - b]).wait()

                @pl.when(i + 1 < nchunk)
                def _fire_next_gather():
                    pltpu.make_async_copy(
                        x_hbm.at[wid, i + 1], ib[1 - b], si[1 - b]).wait()
                    pltpu.async_copy(tok_hbm.at[ib[1 - b]], rnb, sg[1 - b])

                pltpu.make_async_copy(tok_hbm.at[ib[b]], rb, sg[b]).wait()

                @pl.when(i + 2 < nchunk)
                def _fetch_next_idx():
                    pltpu.async_copy(x_hbm.at[wid, i + 2], ib[b], si[b])

                @plsc.parallel_loop(0, maxlen, unroll=2)
                def _add_t(t):
                    p0 = pos_v[t, pl.ds(0, LANES)]
                    p1 = pos_v[t, pl.ds(half, LANES)]
                    for rr in range(SEQS_PER_CHUNK):
                        r = rr * maxlen + t
                        rb[r, pl.ds(0, LANES)] = rb[r, pl.ds(0, LANES)] + p0
                        rb[r, pl.ds(half, LANES)] = (
                            rb[r, pl.ds(half, LANES)] + p1
                        )

                for kq in range(SEQS_PER_CHUNK):
                    pltpu.async_copy(
                        rb.at[pl.ds(kq * maxlen, maxlen)],
                        out_hbm.at[sq0 + i * SEQS_PER_CHUNK + kq], so[b])
            return carry

        lax.fori_loop(0, nchunk // 2, pair_body, 0)
        for kq in range(SEQS_PER_CHUNK):
            pltpu.make_async_copy(
                bufs[1].at[pl.ds(kq * maxlen, maxlen)],
                out_hbm.at[sq0 + (nchunk - 1) * SEQS_PER_CHUNK + kq],
                so[1]).wait()

    return k


def kernel(x, token_table, pos_table):
    batch, maxlen = x.shape
    vocab, embed = token_table.shape
    rows = batch * maxlen
    rpw = rows // NW
    ch = SEQS_PER_CHUNK * maxlen
    xf = x.reshape(NW, rpw // ch, ch).astype(jnp.int32)
    return _make_kernel(batch, maxlen, embed, vocab)(
        xf, token_table, pos_table
    )
